# baseline (speedup 1.0000x reference)
import functools
import math

import jax
import jax.numpy as jnp
from jax.experimental import pallas as pl
from jax.experimental.pallas import tpu as pltpu

_NUM_EXPERTS = 16
_TOKENS = 2048
_D_MODEL = 2048
_CAPACITY = max(int(math.ceil(_TOKENS / _NUM_EXPERTS * 1.0 * 2.0)), 4)
_SBLK = 256
_NBLK = _TOKENS // _SBLK


def _router_kernel(x_ref, wg_ref, laux_ref, combine_ref, dispatch_ref, rt_ref):
    i = pl.program_id(0)
    S, E, C = _TOKENS, _NUM_EXPERTS, _CAPACITY

    @pl.when(i == 0)
    def _gating():
        logits = jnp.dot(x_ref[...], wg_ref[...],
                         preferred_element_type=jnp.float32)
        m = jnp.max(logits, axis=1, keepdims=True)
        p = jnp.exp(logits - m)
        gates = p / jnp.sum(p, axis=1, keepdims=True)

        iota_e = jax.lax.broadcasted_iota(jnp.int32, (S, E), 1)
        e1 = jnp.argmax(gates, axis=1).astype(jnp.int32)
        mask1 = iota_e == e1[:, None]
        gates_m = jnp.where(mask1, -1.0, gates)
        e2 = jnp.argmax(gates_m, axis=1).astype(jnp.int32)
        mask2 = iota_e == e2[:, None]

        m1f = mask1.astype(jnp.float32)
        m2f = mask2.astype(jnp.float32)
        r_iota = jax.lax.broadcasted_iota(jnp.int32, (S, S), 0)
        c_iota = jax.lax.broadcasted_iota(jnp.int32, (S, S), 1)
        tri = (r_iota >= c_iota).astype(jnp.bfloat16)
        m12 = jnp.concatenate([mask1.astype(jnp.bfloat16),
                               mask2.astype(jnp.bfloat16)], axis=1)
        cums = jnp.dot(tri, m12, preferred_element_type=jnp.float32)
        loc1 = cums[:, :E] - 1.0
        cnt1 = cums[S - 1:S, :E]
        loc2 = cums[:, E:] - 1.0 + cnt1

        me = jnp.mean(gates, axis=0, keepdims=True)
        ce = jnp.mean(m1f, axis=0, keepdims=True)
        laux_ref[0, 0] = jnp.sum(me * ce) * jnp.float32(E)

        keep1 = m1f * (loc1 < C).astype(jnp.float32)
        keep2 = m2f * (loc2 < C).astype(jnp.float32)
        c1 = jnp.sum(loc1 * keep1, axis=1, keepdims=True)
        c2 = jnp.sum(loc2 * keep2, axis=1, keepdims=True)
        g1 = jnp.max(gates * keep1, axis=1, keepdims=True)
        g2 = jnp.max(gates * keep2, axis=1, keepdims=True)
        denom = jnp.maximum(g1 + g2, jnp.finfo(jnp.float32).eps)
        w1 = g1 / denom
        w2 = g2 / denom

        rt_ref[...] = jnp.concatenate(
            [w1, w2, c1, c2,
             e1[:, None].astype(jnp.float32), e2[:, None].astype(jnp.float32),
             jnp.zeros((S, 2), jnp.float32)], axis=1)

    rt = rt_ref[pl.ds(i * _SBLK, _SBLK), :]
    w1 = rt[:, 0:1]
    w2 = rt[:, 1:2]
    c1 = rt[:, 2:3]
    c2 = rt[:, 3:4]
    e1 = rt[:, 4:5]
    e2 = rt[:, 5:6]
    iota_e = jax.lax.broadcasted_iota(jnp.int32, (_SBLK, E), 1).astype(jnp.float32)
    iota_c = jax.lax.broadcasted_iota(jnp.int32, (_SBLK, C), 1).astype(jnp.float32)
    we1 = jnp.where(e1 == iota_e, w1, 0.0)
    we2 = jnp.where(e2 == iota_e, w2, 0.0)
    ch1 = (c1 == iota_c).astype(jnp.float32)
    ch2 = (c2 == iota_c).astype(jnp.float32)
    comb = (we1[:, :, None] * ch1[:, None, :]
            + we2[:, :, None] * ch2[:, None, :])
    combine_ref[...] = comb
    dispatch_ref[...] = comb != 0.0


@jax.jit
def kernel(x, wg):
    S, E, C = _TOKENS, _NUM_EXPERTS, _CAPACITY
    laux, combine, dispatch = pl.pallas_call(
        _router_kernel,
        grid=(_NBLK,),
        in_specs=[
            pl.BlockSpec((S, _D_MODEL), lambda i: (0, 0)),
            pl.BlockSpec((_D_MODEL, E), lambda i: (0, 0)),
        ],
        out_specs=[
            pl.BlockSpec((1, 1), lambda i: (0, 0), memory_space=pltpu.SMEM),
            pl.BlockSpec((_SBLK, E, C), lambda i: (i, 0, 0)),
            pl.BlockSpec((_SBLK, E, C), lambda i: (i, 0, 0)),
        ],
        out_shape=[
            jax.ShapeDtypeStruct((1, 1), jnp.float32),
            jax.ShapeDtypeStruct((S, E, C), jnp.float32),
            jax.ShapeDtypeStruct((S, E, C), jnp.bool_),
        ],
        scratch_shapes=[pltpu.VMEM((S, 8), jnp.float32)],
    )(x, wg)
    return (laux[0, 0], combine, dispatch)
